# baseline (speedup 1.0000x reference)
import jax
import jax.numpy as jnp
from jax.experimental import pallas as pl
from jax.experimental.pallas import tpu as pltpu

TILE_IN = 20
HID = 64
OUT = 64
N_ROUNDS = 2
N_HEXES = 19
N_VERTICES = 54
N_EDGES = 72
_NTOT = N_HEXES + N_VERTICES + N_EDGES
_NSRC = N_HEXES + 2 * N_VERTICES + N_EDGES

_TB = 512
_HP = _TB // 2
_BF = jnp.bfloat16


def _avg_mat(adj, mask, n_src):
    oh = (adj[..., None] == jnp.arange(n_src)[None, None, :]).astype(jnp.float32)
    m = mask.astype(jnp.float32)
    a = jnp.sum(oh * m[..., None], axis=1)
    cnt = jnp.clip(jnp.sum(m, axis=1), 1.0, None)
    return a / cnt[:, None]


def _ln_relu(y, g, beta, out_dtype=_BF):
    z = y * y
    seg = jnp.kron(jnp.eye(2, dtype=jnp.float32),
                   jnp.ones((HID, HID), jnp.float32))
    s = jnp.dot(z, seg, preferred_element_type=jnp.float32)
    inv = jax.lax.rsqrt(s * (1.0 / HID) + 1e-5)
    y = y * inv * g[None, :] + beta[None, :]
    return jnp.maximum(y, 0.0).astype(out_dtype)


def _dot(x, w):
    return jnp.dot(x, w, preferred_element_type=jnp.float32)


def _to_wide(x2, n):
    x3 = x2.reshape(n, _HP, 2 * HID)
    return x3.astype(_BF).reshape(n, _TB * HID)


def _body(tf_ref, avh_ref, aev_ref, abig_ref, *rest):
    w_refs = rest[:-1]
    out_ref = rest[-1]
    w = [r[...] for r in w_refs]
    (hw, hb, hg, hbt, vw, vb, vg, vbt, ew, eb, eg, ebt) = w[:12]
    rnd = [w[12 + 12 * r:24 + 12 * r] for r in range(N_ROUNDS)]
    row, rob, rog, robt = w[12 + 12 * N_ROUNDS:]

    tfv = tf_ref[...]
    hex_h = _ln_relu(_dot(tfv.reshape(N_HEXES * _HP, 2 * TILE_IN), hw)
                     + hb[None, :], hg, hbt)

    t_w = tfv.reshape(N_HEXES, _TB * TILE_IN)
    vraw_w = _dot(avh_ref[...], t_w)
    vraw_p = vraw_w.reshape(N_VERTICES, _HP, 2 * TILE_IN).astype(_BF)
    vertex_h = _ln_relu(
        _dot(vraw_p.reshape(N_VERTICES * _HP, 2 * TILE_IN), vw)
        + vb[None, :], vg, vbt)

    eraw_w = _dot(aev_ref[...], vraw_w.astype(_BF))
    eraw_p = eraw_w.reshape(N_EDGES, _HP, 2 * TILE_IN).astype(_BF)
    edge_h = _ln_relu(
        _dot(eraw_p.reshape(N_EDGES * _HP, 2 * TILE_IN), ew)
        + eb[None, :], eg, ebt)

    for r in range(N_ROUNDS):
        whex, wvert, wedge, bh, bv, be = rnd[r][:6]
        hgm, hbm, vgm, vbm, egm, ebm = rnd[r][6:]

        hx = _dot(hex_h, whex)
        vx = _dot(vertex_h, wvert)
        ex = _dot(edge_h, wedge)

        srcw = jnp.concatenate([
            _to_wide(hx[:, 2 * HID:], N_HEXES),
            _to_wide(vx[:, 2 * HID:4 * HID], N_VERTICES),
            _to_wide(vx[:, 4 * HID:], N_VERTICES),
            _to_wide(ex[:, 2 * HID:], N_EDGES),
        ], axis=0)
        gw = _dot(abig_ref[...], srcw)
        g3 = gw.reshape(_NTOT, _HP, 2 * HID)

        hex_h = _ln_relu(
            hx[:, :2 * HID] + g3[0:N_HEXES].reshape(N_HEXES * _HP, 2 * HID)
            + bh[None, :], hgm, hbm)
        vertex_h = _ln_relu(
            vx[:, :2 * HID] + g3[N_HEXES:73].reshape(N_VERTICES * _HP, 2 * HID)
            + bv[None, :], vgm, vbm)
        edge_h = _ln_relu(
            ex[:, :2 * HID] + g3[73:_NTOT].reshape(N_EDGES * _HP, 2 * HID)
            + be[None, :], egm, ebm)

    mh = jnp.mean(hex_h.reshape(N_HEXES, _HP, 2 * HID).astype(jnp.float32),
                  axis=0).astype(_BF)
    mv = jnp.mean(vertex_h.reshape(N_VERTICES, _HP, 2 * HID).astype(jnp.float32),
                  axis=0).astype(_BF)
    me = jnp.mean(edge_h.reshape(N_EDGES, _HP, 2 * HID).astype(jnp.float32),
                  axis=0).astype(_BF)
    ro = _ln_relu(_dot(mh, row[:, :2 * HID]) + _dot(mv, row[:, 2 * HID:4 * HID])
                  + _dot(me, row[:, 4 * HID:]) + rob[None, :],
                  rog, robt, out_dtype=jnp.float32)
    out_ref[...] = ro


def kernel(tile_features, params, hex_to_vertex, vertex_to_hex, edge_to_vertex,
           vertex_to_edge, h2v_mask, v2h_mask, e2v_mask, v2e_mask):
    b = tile_features.shape[0]
    f32 = jnp.float32
    a_vh = _avg_mat(vertex_to_hex, v2h_mask, N_HEXES)
    a_ev = _avg_mat(edge_to_vertex, e2v_mask, N_VERTICES)
    a_hv = _avg_mat(hex_to_vertex, h2v_mask, N_VERTICES)
    a_ve = _avg_mat(vertex_to_edge, v2e_mask, N_EDGES)

    abig = jnp.zeros((_NTOT, _NSRC), f32)
    abig = abig.at[0:N_HEXES, N_HEXES:73].set(a_hv)
    abig = abig.at[N_HEXES:73, 0:N_HEXES].set(a_vh)
    abig = abig.at[N_HEXES:73, 127:_NSRC].set(a_ve)
    abig = abig.at[73:_NTOT, 73:127].set(a_ev)

    tf_t = jnp.transpose(tile_features, (1, 0, 2)).astype(_BF)
    tf_t = tf_t.reshape(N_HEXES, b // 2, 2 * TILE_IN)

    cmat = jnp.eye(HID, dtype=f32) - 1.0 / HID
    eye2 = jnp.eye(2, dtype=f32)
    bf = lambda x: x.astype(_BF)
    pw = lambda x: bf(jnp.kron(eye2, x @ cmat))
    pb = lambda x: jnp.tile(x @ cmat, 2)
    pp = lambda x: jnp.tile(x, 2)

    weights = []
    for name in ('hex_in', 'vertex_in', 'edge_in'):
        wm, bias, g, beta = params[name]
        weights.extend([pw(wm), pb(bias), pp(g), pp(beta)])
    for r in range(N_ROUNDS):
        hwW, hbias, hg, hbt = params['hex_up'][r]
        vwW, vbias, vg, vbt = params['vertex_up'][r]
        ewW, ebias, eg, ebt = params['edge_up'][r]
        whex = jnp.concatenate([pw(hwW[:HID]), pw(vwW[HID:2 * HID])], axis=1)
        wvert = jnp.concatenate([pw(vwW[:HID]), pw(hwW[HID:]), pw(ewW[HID:])],
                                axis=1)
        wedge = jnp.concatenate([pw(ewW[:HID]), pw(vwW[2 * HID:])], axis=1)
        weights.extend([whex, wvert, wedge, pb(hbias), pb(vbias), pb(ebias),
                        pp(hg), pp(hbt), pp(vg), pp(vbt), pp(eg), pp(ebt)])
    rw, rbias, rg, rbt = params['readout']
    rowc = jnp.concatenate([pw(rw[:HID]), pw(rw[HID:2 * HID]),
                            pw(rw[2 * HID:])], axis=1)
    weights.extend([rowc, pb(rbias), pp(rg), pp(rbt)])

    full = lambda arr: pl.BlockSpec(arr.shape, lambda i: (0,) * arr.ndim)
    in_specs = [
        pl.BlockSpec((N_HEXES, _HP, 2 * TILE_IN), lambda i: (0, i, 0)),
        full(a_vh), full(a_ev), full(abig),
    ] + [full(w) for w in weights]

    out = pl.pallas_call(
        _body,
        grid=(b // _TB,),
        in_specs=in_specs,
        out_specs=pl.BlockSpec((_HP, 2 * OUT), lambda i: (i, 0)),
        out_shape=jax.ShapeDtypeStruct((b // 2, 2 * OUT), jnp.float32),
        compiler_params=pltpu.CompilerParams(
            dimension_semantics=("arbitrary",),
        ),
    )(tf_t, bf(a_vh), bf(a_ev), bf(abig), *weights)
    return out.reshape(b, OUT)


# speedup vs baseline: 1.1700x; 1.1700x over previous
import jax
import jax.numpy as jnp
from jax.experimental import pallas as pl
from jax.experimental.pallas import tpu as pltpu

TILE_IN = 20
HID = 64
OUT = 64
N_ROUNDS = 2
N_HEXES = 19
N_VERTICES = 54
N_EDGES = 72
_NTOT = N_HEXES + N_VERTICES + N_EDGES
_NSRC = N_HEXES + 2 * N_VERTICES + N_EDGES

_TB = 128
_HP = _TB // 2
_BF = jnp.bfloat16


def _avg_mat(adj, mask, n_src):
    oh = (adj[..., None] == jnp.arange(n_src)[None, None, :]).astype(jnp.float32)
    m = mask.astype(jnp.float32)
    a = jnp.sum(oh * m[..., None], axis=1)
    cnt = jnp.clip(jnp.sum(m, axis=1), 1.0, None)
    return a / cnt[:, None]


def _ln_relu(y, g, beta, out_dtype=_BF):
    z = y * y
    seg = jnp.kron(jnp.eye(2, dtype=jnp.float32),
                   jnp.ones((HID, HID), jnp.float32))
    s = jnp.dot(z, seg, preferred_element_type=jnp.float32)
    inv = jax.lax.rsqrt(s * (1.0 / HID) + 1e-5)
    y = y * inv * g[None, :] + beta[None, :]
    return jnp.maximum(y, 0.0).astype(out_dtype)


def _dot(x, w):
    return jnp.dot(x, w, preferred_element_type=jnp.float32)


def _to_wide(x2, n):
    x3 = x2.reshape(n, _HP, 2 * HID)
    return x3.astype(_BF).reshape(n, _TB * HID)


def _body(tf_ref, avh_ref, aev_ref, abig_ref, *rest):
    w_refs = rest[:-1]
    out_ref = rest[-1]
    w = [r[...] for r in w_refs]
    (hw, hb, hg, hbt, vw, vb, vg, vbt, ew, eb, eg, ebt) = w[:12]
    rnd = [w[12 + 12 * r:24 + 12 * r] for r in range(N_ROUNDS)]
    row, rob, rog, robt = w[12 + 12 * N_ROUNDS:]

    tfv = tf_ref[...]
    hex_h = _ln_relu(_dot(tfv.reshape(N_HEXES * _HP, 2 * TILE_IN), hw)
                     + hb[None, :], hg, hbt)

    t_w = tfv.reshape(N_HEXES, _TB * TILE_IN)
    vraw_w = _dot(avh_ref[...], t_w)
    vraw_p = vraw_w.reshape(N_VERTICES, _HP, 2 * TILE_IN).astype(_BF)
    vertex_h = _ln_relu(
        _dot(vraw_p.reshape(N_VERTICES * _HP, 2 * TILE_IN), vw)
        + vb[None, :], vg, vbt)

    eraw_w = _dot(aev_ref[...], vraw_w.astype(_BF))
    eraw_p = eraw_w.reshape(N_EDGES, _HP, 2 * TILE_IN).astype(_BF)
    edge_h = _ln_relu(
        _dot(eraw_p.reshape(N_EDGES * _HP, 2 * TILE_IN), ew)
        + eb[None, :], eg, ebt)

    for r in range(N_ROUNDS):
        whex, wvert, wedge, bh, bv, be = rnd[r][:6]
        hgm, hbm, vgm, vbm, egm, ebm = rnd[r][6:]

        hx = _dot(hex_h, whex)
        vx = _dot(vertex_h, wvert)
        ex = _dot(edge_h, wedge)

        srcw = jnp.concatenate([
            _to_wide(hx[:, 2 * HID:], N_HEXES),
            _to_wide(vx[:, 2 * HID:4 * HID], N_VERTICES),
            _to_wide(vx[:, 4 * HID:], N_VERTICES),
            _to_wide(ex[:, 2 * HID:], N_EDGES),
        ], axis=0)
        gw = _dot(abig_ref[...], srcw)
        g3 = gw.reshape(_NTOT, _HP, 2 * HID)

        hex_h = _ln_relu(
            hx[:, :2 * HID] + g3[0:N_HEXES].reshape(N_HEXES * _HP, 2 * HID)
            + bh[None, :], hgm, hbm)
        vertex_h = _ln_relu(
            vx[:, :2 * HID] + g3[N_HEXES:73].reshape(N_VERTICES * _HP, 2 * HID)
            + bv[None, :], vgm, vbm)
        edge_h = _ln_relu(
            ex[:, :2 * HID] + g3[73:_NTOT].reshape(N_EDGES * _HP, 2 * HID)
            + be[None, :], egm, ebm)

    mh = jnp.mean(hex_h.reshape(N_HEXES, _HP, 2 * HID).astype(jnp.float32),
                  axis=0).astype(_BF)
    mv = jnp.mean(vertex_h.reshape(N_VERTICES, _HP, 2 * HID).astype(jnp.float32),
                  axis=0).astype(_BF)
    me = jnp.mean(edge_h.reshape(N_EDGES, _HP, 2 * HID).astype(jnp.float32),
                  axis=0).astype(_BF)
    ro = _ln_relu(_dot(mh, row[:, :2 * HID]) + _dot(mv, row[:, 2 * HID:4 * HID])
                  + _dot(me, row[:, 4 * HID:]) + rob[None, :],
                  rog, robt, out_dtype=jnp.float32)
    out_ref[...] = ro


def kernel(tile_features, params, hex_to_vertex, vertex_to_hex, edge_to_vertex,
           vertex_to_edge, h2v_mask, v2h_mask, e2v_mask, v2e_mask):
    b = tile_features.shape[0]
    f32 = jnp.float32
    a_vh = _avg_mat(vertex_to_hex, v2h_mask, N_HEXES)
    a_ev = _avg_mat(edge_to_vertex, e2v_mask, N_VERTICES)
    a_hv = _avg_mat(hex_to_vertex, h2v_mask, N_VERTICES)
    a_ve = _avg_mat(vertex_to_edge, v2e_mask, N_EDGES)

    abig = jnp.zeros((_NTOT, _NSRC), f32)
    abig = abig.at[0:N_HEXES, N_HEXES:73].set(a_hv)
    abig = abig.at[N_HEXES:73, 0:N_HEXES].set(a_vh)
    abig = abig.at[N_HEXES:73, 127:_NSRC].set(a_ve)
    abig = abig.at[73:_NTOT, 73:127].set(a_ev)

    tf_t = jnp.transpose(tile_features, (1, 0, 2)).astype(_BF)
    tf_t = tf_t.reshape(N_HEXES, b // 2, 2 * TILE_IN)

    cmat = jnp.eye(HID, dtype=f32) - 1.0 / HID
    eye2 = jnp.eye(2, dtype=f32)
    bf = lambda x: x.astype(_BF)
    pw = lambda x: bf(jnp.kron(eye2, x @ cmat))
    pb = lambda x: jnp.tile(x @ cmat, 2)
    pp = lambda x: jnp.tile(x, 2)

    weights = []
    for name in ('hex_in', 'vertex_in', 'edge_in'):
        wm, bias, g, beta = params[name]
        weights.extend([pw(wm), pb(bias), pp(g), pp(beta)])
    for r in range(N_ROUNDS):
        hwW, hbias, hg, hbt = params['hex_up'][r]
        vwW, vbias, vg, vbt = params['vertex_up'][r]
        ewW, ebias, eg, ebt = params['edge_up'][r]
        whex = jnp.concatenate([pw(hwW[:HID]), pw(vwW[HID:2 * HID])], axis=1)
        wvert = jnp.concatenate([pw(vwW[:HID]), pw(hwW[HID:]), pw(ewW[HID:])],
                                axis=1)
        wedge = jnp.concatenate([pw(ewW[:HID]), pw(vwW[2 * HID:])], axis=1)
        weights.extend([whex, wvert, wedge, pb(hbias), pb(vbias), pb(ebias),
                        pp(hg), pp(hbt), pp(vg), pp(vbt), pp(eg), pp(ebt)])
    rw, rbias, rg, rbt = params['readout']
    rowc = jnp.concatenate([pw(rw[:HID]), pw(rw[HID:2 * HID]),
                            pw(rw[2 * HID:])], axis=1)
    weights.extend([rowc, pb(rbias), pp(rg), pp(rbt)])

    full = lambda arr: pl.BlockSpec(arr.shape, lambda i: (0,) * arr.ndim)
    in_specs = [
        pl.BlockSpec((N_HEXES, _HP, 2 * TILE_IN), lambda i: (0, i, 0)),
        full(a_vh), full(a_ev), full(abig),
    ] + [full(w) for w in weights]

    out = pl.pallas_call(
        _body,
        grid=(b // _TB,),
        in_specs=in_specs,
        out_specs=pl.BlockSpec((_HP, 2 * OUT), lambda i: (i, 0)),
        out_shape=jax.ShapeDtypeStruct((b // 2, 2 * OUT), jnp.float32),
        compiler_params=pltpu.CompilerParams(
            dimension_semantics=("arbitrary",),
        ),
    )(tf_t, bf(a_vh), bf(a_ev), bf(abig), *weights)
    return out.reshape(b, OUT)
